# Initial kernel scaffold; baseline (speedup 1.0000x reference)
import jax
import jax.numpy as jnp
from jax import lax
from jax.experimental import pallas as pl
from jax.experimental.pallas import tpu as pltpu
from jax.experimental.pallas import tpu_sc as plsc

N = 10000
D = 128
NNZ = 100000
E = 320000
NC = 2
NS = 16
NW = NC * NS

NNZ_PW = 3136
NNZ_PAD = NNZ_PW * NW
ZCH = 2000
SPAN = (N * D) // NS
ZR = 125
ROWS_PT = N // NS

K_E = 200
E_PW = E // NW
NCH = E_PW // K_E


def _mesh():
    return plsc.VectorSubcoreMesh(core_axis_name="c", subcore_axis_name="s")


def _feat_scatter(fr, fc, fv):
    @pl.kernel(
        out_type=jax.ShapeDtypeStruct((NC, N * D), jnp.float32),
        mesh=_mesh(),
        scratch_types=[
            pltpu.VMEM((NNZ_PW,), jnp.int32),
            pltpu.VMEM((NNZ_PW,), jnp.int32),
            pltpu.VMEM((NNZ_PW,), jnp.int32),
            pltpu.VMEM((NNZ_PW,), jnp.float32),
            pltpu.VMEM((ZCH,), jnp.float32),
            pltpu.VMEM_SHARED((N * D,), jnp.float32),
        ],
    )
    def k(fr_hbm, fc_hbm, fv_hbm, s_hbm, r_v, c_v, i_v, v_v, z_v, s_sh):
        cid = lax.axis_index("c")
        sid = lax.axis_index("s")
        wid = cid * NS + sid
        base = wid * NNZ_PW
        pltpu.sync_copy(fr_hbm.at[pl.ds(base, NNZ_PW)], r_v)
        pltpu.sync_copy(fc_hbm.at[pl.ds(base, NNZ_PW)], c_v)
        pltpu.sync_copy(fv_hbm.at[pl.ds(base, NNZ_PW)], v_v)

        @pl.loop(0, ZCH, step=16)
        def _(i):
            z_v[pl.ds(i, 16)] = jnp.zeros((16,), jnp.float32)

        @pl.loop(0, SPAN, step=ZCH)
        def _(j):
            pltpu.sync_copy(z_v, s_sh.at[pl.ds(sid * SPAN + j, ZCH)])

        @pl.loop(0, NNZ_PW, step=16)
        def _(i):
            sl = pl.ds(i, 16)
            i_v[sl] = r_v[sl] * D + c_v[sl]

        plsc.subcore_barrier()
        pltpu.sync_copy(v_v, s_sh.at[i_v], add=True)
        plsc.subcore_barrier()
        pltpu.sync_copy(s_sh.at[pl.ds(sid * SPAN, SPAN)],
                        s_hbm.at[cid, pl.ds(sid * SPAN, SPAN)])

    return k(fr, fc, fv)


def _edge_pass(src, dst, adj, xw):
    @pl.kernel(
        out_type=jax.ShapeDtypeStruct((NC, N, D), jnp.float32),
        mesh=_mesh(),
        scratch_types=[
            pltpu.VMEM((K_E,), jnp.int32),
            pltpu.VMEM((K_E,), jnp.int32),
            pltpu.VMEM((K_E,), jnp.int32),
            pltpu.VMEM((K_E,), jnp.int32),
            pltpu.SMEM((K_E,), jnp.float32),
            pltpu.SMEM((K_E,), jnp.float32),
            pltpu.VMEM((K_E, D), jnp.float32),
            pltpu.VMEM((K_E, D), jnp.float32),
            pltpu.VMEM((ZR, D), jnp.float32),
            pltpu.VMEM_SHARED((N, D), jnp.float32),
            pltpu.SemaphoreType.DMA,
            pltpu.SemaphoreType.DMA,
        ],
    )
    def k(src_hbm, dst_hbm, adj_hbm, xw_hbm, o_hbm,
          si0, si1, di0, di1, a0, a1, r0, r1, z_v, o_sh, g0, g1):
        cid = lax.axis_index("c")
        sid = lax.axis_index("s")
        wid = cid * NS + sid
        wbase = wid * E_PW

        def start(j, si, di, a, r, g):
            b = wbase + j * K_E
            pltpu.sync_copy(src_hbm.at[pl.ds(b, K_E)], si)
            pltpu.sync_copy(dst_hbm.at[pl.ds(b, K_E)], di)
            pltpu.sync_copy(adj_hbm.at[pl.ds(b, K_E)], a)
            pltpu.async_copy(xw_hbm.at[si], r, g)

        def work(si, di, a, r, g):
            pltpu.make_async_copy(xw_hbm.at[si], r, g).wait()

            @pl.loop(0, K_E)
            def _(kk):
                av = jnp.full((16,), a[kk], jnp.float32)
                for d in range(8):
                    r[kk, pl.ds(d * 16, 16)] = r[kk, pl.ds(d * 16, 16)] * av

            pltpu.sync_copy(r, o_sh.at[di], add=True)

        start(0, si0, di0, a0, r0, g0)
        start(1, si1, di1, a1, r1, g1)

        @pl.loop(0, ZR)
        def _(i):
            for d in range(8):
                z_v[i, pl.ds(d * 16, 16)] = jnp.zeros((16,), jnp.float32)

        @pl.loop(0, ROWS_PT, step=ZR)
        def _(j):
            pltpu.sync_copy(z_v, o_sh.at[pl.ds(sid * ROWS_PT + j, ZR)])

        plsc.subcore_barrier()

        @pl.loop(0, NCH, step=2)
        def _(j):
            work(si0, di0, a0, r0, g0)

            @pl.when(j + 2 < NCH)
            def _():
                start(j + 2, si0, di0, a0, r0, g0)

            work(si1, di1, a1, r1, g1)

            @pl.when(j + 3 < NCH)
            def _():
                start(j + 3, si1, di1, a1, r1, g1)

        plsc.subcore_barrier()
        pltpu.sync_copy(o_sh.at[pl.ds(sid * ROWS_PT, ROWS_PT)],
                        o_hbm.at[cid, pl.ds(sid * ROWS_PT, ROWS_PT)])

    return k(src, dst, adj, xw)


BM = 2000


def _mm(s0, s1, w):
    def body(s0_ref, s1_ref, w_ref, o_ref):
        x = s0_ref[...] + s1_ref[...]
        o_ref[...] = jnp.dot(x, w_ref[...], preferred_element_type=jnp.float32)

    return pl.pallas_call(
        body,
        grid=(N // BM,),
        in_specs=[pl.BlockSpec((BM, D), lambda i: (i, 0)),
                  pl.BlockSpec((BM, D), lambda i: (i, 0)),
                  pl.BlockSpec((D, D), lambda i: (0, 0))],
        out_specs=pl.BlockSpec((BM, D), lambda i: (i, 0)),
        out_shape=jax.ShapeDtypeStruct((N, D), jnp.float32),
    )(s0, s1, w)


def _merge_relu(o0, o1):
    def body(a_ref, b_ref, o_ref):
        o_ref[...] = jnp.maximum(a_ref[...] + b_ref[...], 0.0)

    return pl.pallas_call(
        body,
        grid=(N // BM,),
        in_specs=[pl.BlockSpec((BM, D), lambda i: (i, 0)),
                  pl.BlockSpec((BM, D), lambda i: (i, 0))],
        out_specs=pl.BlockSpec((BM, D), lambda i: (i, 0)),
        out_shape=jax.ShapeDtypeStruct((N, D), jnp.float32),
    )(o0, o1)


def kernel(feat_rows, feat_cols, feat_values, edge_index, adj_values, W):
    pad = NNZ_PAD - NNZ
    fr = jnp.concatenate([feat_rows.astype(jnp.int32),
                          jnp.zeros((pad,), jnp.int32)])
    fc = jnp.concatenate([feat_cols.astype(jnp.int32),
                          jnp.zeros((pad,), jnp.int32)])
    fv = jnp.concatenate([feat_values, jnp.zeros((pad,), jnp.float32)])

    s_parts = _feat_scatter(fr, fc, fv)
    xw = _mm(s_parts[0].reshape(N, D), s_parts[1].reshape(N, D), W)

    o_parts = _edge_pass(edge_index[1].astype(jnp.int32),
                         edge_index[0].astype(jnp.int32),
                         adj_values, xw)
    return _merge_relu(o_parts[0], o_parts[1])


# speedup vs baseline: 7.0925x; 7.0925x over previous
import jax
import jax.numpy as jnp
from jax import lax
from jax.experimental import pallas as pl
from jax.experimental.pallas import tpu as pltpu
from jax.experimental.pallas import tpu_sc as plsc

N = 10000
D = 128
NNZ = 100000
E = 320000
NC = 2
NS = 16
NW = NC * NS

NNZ_PW = 3136
NNZ_PAD = NNZ_PW * NW
ZCH = 2000
SPAN = (N * D) // NS
ZR = 16
ROWS_PT = 624
ROWS_REM = N - ROWS_PT * NS

K_E = 128
NCH = 79
E_PW = K_E * NCH
E_PAD = E_PW * NW


def _mesh():
    return plsc.VectorSubcoreMesh(core_axis_name="c", subcore_axis_name="s")


def _feat_scatter(fr, fc, fv):
    @pl.kernel(
        out_type=jax.ShapeDtypeStruct((NC * N * D,), jnp.float32),
        mesh=_mesh(),
        scratch_types=[
            pltpu.VMEM((NNZ_PW,), jnp.int32),
            pltpu.VMEM((NNZ_PW,), jnp.int32),
            pltpu.VMEM((NNZ_PW,), jnp.int32),
            pltpu.VMEM((NNZ_PW,), jnp.float32),
            pltpu.VMEM((ZCH,), jnp.float32),
            pltpu.VMEM_SHARED((N * D,), jnp.float32),
        ],
    )
    def k(fr_hbm, fc_hbm, fv_hbm, s_hbm, r_v, c_v, i_v, v_v, z_v, s_sh):
        cid = lax.axis_index("c")
        sid = lax.axis_index("s")
        wid = cid * NS + sid
        base = wid * NNZ_PW
        pltpu.sync_copy(fr_hbm.at[pl.ds(base, NNZ_PW)], r_v)
        pltpu.sync_copy(fc_hbm.at[pl.ds(base, NNZ_PW)], c_v)
        pltpu.sync_copy(fv_hbm.at[pl.ds(base, NNZ_PW)], v_v)

        @pl.loop(0, ZCH, step=16)
        def _(i):
            z_v[pl.ds(i, 16)] = jnp.zeros((16,), jnp.float32)

        @pl.loop(0, SPAN, step=ZCH)
        def _(j):
            pltpu.sync_copy(z_v, s_sh.at[pl.ds(sid * SPAN + j, ZCH)])

        @pl.loop(0, NNZ_PW, step=16)
        def _(i):
            sl = pl.ds(i, 16)
            i_v[sl] = r_v[sl] * D + c_v[sl]

        plsc.subcore_barrier()
        pltpu.sync_copy(v_v, s_sh.at[i_v], add=True)
        plsc.subcore_barrier()
        pltpu.sync_copy(s_sh.at[pl.ds(sid * SPAN, SPAN)],
                        s_hbm.at[pl.ds(cid * (N * D) + sid * SPAN, SPAN)])

    return k(fr, fc, fv)


def _edge_pass(src, dst, adj, xw):
    @pl.kernel(
        out_type=jax.ShapeDtypeStruct((NC, N, D), jnp.float32),
        mesh=_mesh(),
        scratch_types=[
            pltpu.VMEM((K_E,), jnp.int32),
            pltpu.VMEM((K_E,), jnp.int32),
            pltpu.VMEM((K_E,), jnp.int32),
            pltpu.VMEM((K_E,), jnp.int32),
            pltpu.VMEM((K_E,), jnp.float32),
            pltpu.VMEM((K_E,), jnp.float32),
            pltpu.VMEM((K_E, D), jnp.float32),
            pltpu.VMEM((K_E, D), jnp.float32),
            pltpu.VMEM((ZR, D), jnp.float32),
            pltpu.VMEM_SHARED((N, D), jnp.float32),
            pltpu.SemaphoreType.DMA,
            pltpu.SemaphoreType.DMA,
        ],
    )
    def k(src_hbm, dst_hbm, adj_hbm, xw_hbm, o_hbm,
          si0, si1, di0, di1, a0, a1, r0, r1, z_v, o_sh, g0, g1):
        cid = lax.axis_index("c")
        sid = lax.axis_index("s")
        wid = cid * NS + sid
        wbase = wid * E_PW

        def start(j, si, di, a, r, g):
            b = wbase + j * K_E
            pltpu.sync_copy(src_hbm.at[pl.ds(b, K_E)], si)
            pltpu.sync_copy(dst_hbm.at[pl.ds(b, K_E)], di)
            pltpu.sync_copy(adj_hbm.at[pl.ds(b, K_E)], a)
            pltpu.async_copy(xw_hbm.at[si], r, g)

        def work(si, di, a, r, g):
            pltpu.make_async_copy(xw_hbm.at[si], r, g).wait()

            @pl.loop(0, K_E, step=16)
            def _(gg):
                a_vec = a[pl.ds(gg, 16)]
                for kk in range(16):
                    av = lax.gather(
                        a_vec, jnp.full((16, 1), kk, jnp.int32),
                        lax.GatherDimensionNumbers(
                            offset_dims=(), collapsed_slice_dims=(0,),
                            start_index_map=(0,)),
                        slice_sizes=(1,),
                        mode=lax.GatherScatterMode.PROMISE_IN_BOUNDS)
                    for d in range(8):
                        sl = pl.ds(d * 16, 16)
                        r[gg + kk, sl] = r[gg + kk, sl] * av

            pltpu.sync_copy(r, o_sh.at[di], add=True)

        start(0, si0, di0, a0, r0, g0)
        start(1, si1, di1, a1, r1, g1)

        @pl.loop(0, ZR)
        def _(i):
            for d in range(8):
                z_v[i, pl.ds(d * 16, 16)] = jnp.zeros((16,), jnp.float32)

        @pl.loop(0, ROWS_PT, step=ZR)
        def _(j):
            pltpu.sync_copy(z_v, o_sh.at[pl.ds(sid * ROWS_PT + j, ZR)])

        @pl.when(sid == 0)
        def _():
            pltpu.sync_copy(z_v.at[pl.ds(0, ROWS_REM)],
                            o_sh.at[pl.ds(NS * ROWS_PT, ROWS_REM)])

        plsc.subcore_barrier()

        @pl.loop(0, NCH - 1, step=2)
        def _(j):
            work(si0, di0, a0, r0, g0)

            @pl.when(j + 2 < NCH)
            def _():
                start(j + 2, si0, di0, a0, r0, g0)

            work(si1, di1, a1, r1, g1)

            @pl.when(j + 3 < NCH)
            def _():
                start(j + 3, si1, di1, a1, r1, g1)

        work(si0, di0, a0, r0, g0)

        plsc.subcore_barrier()
        pltpu.sync_copy(o_sh.at[pl.ds(sid * ROWS_PT, ROWS_PT)],
                        o_hbm.at[cid, pl.ds(sid * ROWS_PT, ROWS_PT)])

        @pl.when(sid == 0)
        def _():
            pltpu.sync_copy(o_sh.at[pl.ds(NS * ROWS_PT, ROWS_REM)],
                            o_hbm.at[cid, pl.ds(NS * ROWS_PT, ROWS_REM)])

    return k(src, dst, adj, xw)


BM = 2000


def _mm(s0, s1, w):
    def body(s0_ref, s1_ref, w_ref, o_ref):
        x = s0_ref[...] + s1_ref[...]
        o_ref[...] = jnp.dot(x, w_ref[...], preferred_element_type=jnp.float32)

    return pl.pallas_call(
        body,
        grid=(N // BM,),
        in_specs=[pl.BlockSpec((BM, D), lambda i: (i, 0)),
                  pl.BlockSpec((BM, D), lambda i: (i, 0)),
                  pl.BlockSpec((D, D), lambda i: (0, 0))],
        out_specs=pl.BlockSpec((BM, D), lambda i: (i, 0)),
        out_shape=jax.ShapeDtypeStruct((N, D), jnp.float32),
    )(s0, s1, w)


def _merge_relu(o0, o1):
    def body(a_ref, b_ref, o_ref):
        o_ref[...] = jnp.maximum(a_ref[...] + b_ref[...], 0.0)

    return pl.pallas_call(
        body,
        grid=(N // BM,),
        in_specs=[pl.BlockSpec((BM, D), lambda i: (i, 0)),
                  pl.BlockSpec((BM, D), lambda i: (i, 0))],
        out_specs=pl.BlockSpec((BM, D), lambda i: (i, 0)),
        out_shape=jax.ShapeDtypeStruct((N, D), jnp.float32),
    )(o0, o1)


def kernel(feat_rows, feat_cols, feat_values, edge_index, adj_values, W):
    pad = NNZ_PAD - NNZ
    fr = jnp.concatenate([feat_rows.astype(jnp.int32),
                          jnp.zeros((pad,), jnp.int32)])
    fc = jnp.concatenate([feat_cols.astype(jnp.int32),
                          jnp.zeros((pad,), jnp.int32)])
    fv = jnp.concatenate([feat_values, jnp.zeros((pad,), jnp.float32)])

    s_parts = _feat_scatter(fr, fc, fv).reshape(NC, N, D)
    xw = _mm(s_parts[0], s_parts[1], W)

    epad = E_PAD - E
    pad_idx = (jnp.arange(epad, dtype=jnp.int32) % N)
    src = jnp.concatenate([edge_index[1].astype(jnp.int32), pad_idx])
    dst = jnp.concatenate([edge_index[0].astype(jnp.int32), pad_idx])
    adj = jnp.concatenate([adj_values, jnp.zeros((epad,), jnp.float32)])

    o_parts = _edge_pass(src, dst, adj, xw)
    return _merge_relu(o_parts[0], o_parts[1])


# speedup vs baseline: 7.8270x; 1.1036x over previous
import jax
import jax.numpy as jnp
from jax import lax
from jax.experimental import pallas as pl
from jax.experimental.pallas import tpu as pltpu
from jax.experimental.pallas import tpu_sc as plsc

N = 10000
D = 128
NNZ = 100000
E = 320000
NC = 2
NS = 16
NW = NC * NS

NNZ_PW = 3136
NNZ_PAD = NNZ_PW * NW
ZCH = 2000
SPAN = (N * D) // NS
ZR = 16
ROWS_PT = 624
ROWS_REM = N - ROWS_PT * NS

K_E = 112
NCH = 90
E_PW = K_E * NCH
E_PAD = E_PW * NW


def _mesh():
    return plsc.VectorSubcoreMesh(core_axis_name="c", subcore_axis_name="s")


def _feat_scatter(fr, fc, fv):
    @pl.kernel(
        out_type=jax.ShapeDtypeStruct((NC * N * D,), jnp.float32),
        mesh=_mesh(),
        scratch_types=[
            pltpu.VMEM((NNZ_PW,), jnp.int32),
            pltpu.VMEM((NNZ_PW,), jnp.int32),
            pltpu.VMEM((NNZ_PW,), jnp.int32),
            pltpu.VMEM((NNZ_PW,), jnp.float32),
            pltpu.VMEM((ZCH,), jnp.float32),
            pltpu.VMEM_SHARED((N * D,), jnp.float32),
        ],
    )
    def k(fr_hbm, fc_hbm, fv_hbm, s_hbm, r_v, c_v, i_v, v_v, z_v, s_sh):
        cid = lax.axis_index("c")
        sid = lax.axis_index("s")
        wid = cid * NS + sid
        base = wid * NNZ_PW
        pltpu.sync_copy(fr_hbm.at[pl.ds(base, NNZ_PW)], r_v)
        pltpu.sync_copy(fc_hbm.at[pl.ds(base, NNZ_PW)], c_v)
        pltpu.sync_copy(fv_hbm.at[pl.ds(base, NNZ_PW)], v_v)

        @pl.loop(0, ZCH, step=16)
        def _(i):
            z_v[pl.ds(i, 16)] = jnp.zeros((16,), jnp.float32)

        @pl.loop(0, SPAN, step=ZCH)
        def _(j):
            pltpu.sync_copy(z_v, s_sh.at[pl.ds(sid * SPAN + j, ZCH)])

        @pl.loop(0, NNZ_PW, step=16)
        def _(i):
            sl = pl.ds(i, 16)
            i_v[sl] = r_v[sl] * D + c_v[sl]

        plsc.subcore_barrier()
        pltpu.sync_copy(v_v, s_sh.at[i_v], add=True)
        plsc.subcore_barrier()
        pltpu.sync_copy(s_sh.at[pl.ds(sid * SPAN, SPAN)],
                        s_hbm.at[pl.ds(cid * (N * D) + sid * SPAN, SPAN)])

    return k(fr, fc, fv)


def _edge_pass(src, dst, adj, xw):
    @pl.kernel(
        out_type=jax.ShapeDtypeStruct((NC, N, D), jnp.float32),
        mesh=_mesh(),
        scratch_types=(
            [pltpu.VMEM((K_E,), jnp.int32)] * 6
            + [pltpu.VMEM((K_E,), jnp.float32)] * 3
            + [pltpu.VMEM((K_E, D), jnp.float32)] * 3
            + [pltpu.VMEM((ZR, D), jnp.float32),
               pltpu.VMEM_SHARED((N, D), jnp.float32)]
            + [pltpu.SemaphoreType.DMA] * 6
        ),
    )
    def k(src_hbm, dst_hbm, adj_hbm, xw_hbm, o_hbm,
          si0, si1, si2, di0, di1, di2, a0, a1, a2, r0, r1, r2,
          z_v, o_sh, g0, g1, g2, s0, s1, s2):
        cid = lax.axis_index("c")
        sid = lax.axis_index("s")
        wid = cid * NS + sid
        wbase = wid * E_PW

        def start(j, si, di, a, r, g):
            b = wbase + j * K_E
            pltpu.sync_copy(src_hbm.at[pl.ds(b, K_E)], si)
            pltpu.sync_copy(dst_hbm.at[pl.ds(b, K_E)], di)
            pltpu.sync_copy(adj_hbm.at[pl.ds(b, K_E)], a)
            pltpu.async_copy(xw_hbm.at[si], r, g)

        def drain_scatter(di, r, s):
            pltpu.make_async_copy(r, o_sh.at[di], s).wait()

        def work(si, di, a, r, g, s):
            pltpu.make_async_copy(xw_hbm.at[si], r, g).wait()

            @pl.loop(0, K_E, step=16)
            def _(gg):
                a_vec = a[pl.ds(gg, 16)]
                for kk in range(16):
                    av = lax.gather(
                        a_vec, jnp.full((16, 1), kk, jnp.int32),
                        lax.GatherDimensionNumbers(
                            offset_dims=(), collapsed_slice_dims=(0,),
                            start_index_map=(0,)),
                        slice_sizes=(1,),
                        mode=lax.GatherScatterMode.PROMISE_IN_BOUNDS)
                    for d in range(8):
                        sl = pl.ds(d * 16, 16)
                        r[gg + kk, sl] = r[gg + kk, sl] * av

            pltpu.async_copy(r, o_sh.at[di], s, add=True)

        start(0, si0, di0, a0, r0, g0)
        start(1, si1, di1, a1, r1, g1)
        start(2, si2, di2, a2, r2, g2)

        @pl.loop(0, ZR)
        def _(i):
            for d in range(8):
                z_v[i, pl.ds(d * 16, 16)] = jnp.zeros((16,), jnp.float32)

        @pl.loop(0, ROWS_PT, step=ZR)
        def _(j):
            pltpu.sync_copy(z_v, o_sh.at[pl.ds(sid * ROWS_PT + j, ZR)])

        @pl.when(sid == 0)
        def _():
            pltpu.sync_copy(z_v.at[pl.ds(0, ROWS_REM)],
                            o_sh.at[pl.ds(NS * ROWS_PT, ROWS_REM)])

        plsc.subcore_barrier()

        @pl.loop(0, NCH, step=3)
        def _(j):
            work(si0, di0, a0, r0, g0, s0)
            work(si1, di1, a1, r1, g1, s1)
            drain_scatter(di0, r0, s0)

            @pl.when(j + 3 < NCH)
            def _():
                start(j + 3, si0, di0, a0, r0, g0)

            work(si2, di2, a2, r2, g2, s2)
            drain_scatter(di1, r1, s1)

            @pl.when(j + 4 < NCH)
            def _():
                start(j + 4, si1, di1, a1, r1, g1)

            drain_scatter(di2, r2, s2)

            @pl.when(j + 5 < NCH)
            def _():
                start(j + 5, si2, di2, a2, r2, g2)

        plsc.subcore_barrier()
        pltpu.sync_copy(o_sh.at[pl.ds(sid * ROWS_PT, ROWS_PT)],
                        o_hbm.at[cid, pl.ds(sid * ROWS_PT, ROWS_PT)])

        @pl.when(sid == 0)
        def _():
            pltpu.sync_copy(o_sh.at[pl.ds(NS * ROWS_PT, ROWS_REM)],
                            o_hbm.at[cid, pl.ds(NS * ROWS_PT, ROWS_REM)])

    return k(src, dst, adj, xw)


BM = 2000


def _mm(s0, s1, w):
    def body(s0_ref, s1_ref, w_ref, o_ref):
        x = s0_ref[...] + s1_ref[...]
        o_ref[...] = jnp.dot(x, w_ref[...], preferred_element_type=jnp.float32)

    return pl.pallas_call(
        body,
        grid=(N // BM,),
        in_specs=[pl.BlockSpec((BM, D), lambda i: (i, 0)),
                  pl.BlockSpec((BM, D), lambda i: (i, 0)),
                  pl.BlockSpec((D, D), lambda i: (0, 0))],
        out_specs=pl.BlockSpec((BM, D), lambda i: (i, 0)),
        out_shape=jax.ShapeDtypeStruct((N, D), jnp.float32),
    )(s0, s1, w)


def _merge_relu(o0, o1):
    def body(a_ref, b_ref, o_ref):
        o_ref[...] = jnp.maximum(a_ref[...] + b_ref[...], 0.0)

    return pl.pallas_call(
        body,
        grid=(N // BM,),
        in_specs=[pl.BlockSpec((BM, D), lambda i: (i, 0)),
                  pl.BlockSpec((BM, D), lambda i: (i, 0))],
        out_specs=pl.BlockSpec((BM, D), lambda i: (i, 0)),
        out_shape=jax.ShapeDtypeStruct((N, D), jnp.float32),
    )(o0, o1)


def kernel(feat_rows, feat_cols, feat_values, edge_index, adj_values, W):
    pad = NNZ_PAD - NNZ
    fr = jnp.concatenate([feat_rows.astype(jnp.int32),
                          jnp.zeros((pad,), jnp.int32)])
    fc = jnp.concatenate([feat_cols.astype(jnp.int32),
                          jnp.zeros((pad,), jnp.int32)])
    fv = jnp.concatenate([feat_values, jnp.zeros((pad,), jnp.float32)])

    s_parts = _feat_scatter(fr, fc, fv).reshape(NC, N, D)
    xw = _mm(s_parts[0], s_parts[1], W)

    epad = E_PAD - E
    pad_idx = (jnp.arange(epad, dtype=jnp.int32) % N)
    src = jnp.concatenate([edge_index[1].astype(jnp.int32), pad_idx])
    dst = jnp.concatenate([edge_index[0].astype(jnp.int32), pad_idx])
    adj = jnp.concatenate([adj_values, jnp.zeros((epad,), jnp.float32)])

    o_parts = _edge_pass(src, dst, adj, xw)
    return _merge_relu(o_parts[0], o_parts[1])


# speedup vs baseline: 9.7727x; 1.2486x over previous
import jax
import jax.numpy as jnp
from jax import lax
from jax.experimental import pallas as pl
from jax.experimental.pallas import tpu as pltpu
from jax.experimental.pallas import tpu_sc as plsc

N = 10000
D = 128
NNZ = 100000
E = 320000
NC = 2
NS = 16
NW = NC * NS

NNZ_PW = 3136
NNZ_PAD = NNZ_PW * NW
ZCH = 2000
SPAN = (N * D) // NS
ZR = 16
ROWS_PT = 624
ROWS_REM = N - ROWS_PT * NS

K_E = 112
NCH = 90
E_PW = K_E * NCH
E_PAD = E_PW * NW


def _mesh():
    return plsc.VectorSubcoreMesh(core_axis_name="c", subcore_axis_name="s")


def _feat_scatter(fr, fc, fv):
    @pl.kernel(
        out_type=jax.ShapeDtypeStruct((NC * N * D,), jnp.float32),
        mesh=_mesh(),
        scratch_types=[
            pltpu.VMEM((NNZ_PW,), jnp.int32),
            pltpu.VMEM((NNZ_PW,), jnp.int32),
            pltpu.VMEM((NNZ_PW,), jnp.int32),
            pltpu.VMEM((NNZ_PW,), jnp.float32),
            pltpu.VMEM((ZCH,), jnp.float32),
            pltpu.VMEM_SHARED((N * D,), jnp.float32),
        ],
    )
    def k(fr_hbm, fc_hbm, fv_hbm, s_hbm, r_v, c_v, i_v, v_v, z_v, s_sh):
        cid = lax.axis_index("c")
        sid = lax.axis_index("s")
        wid = cid * NS + sid
        base = wid * NNZ_PW
        pltpu.sync_copy(fr_hbm.at[pl.ds(base, NNZ_PW)], r_v)
        pltpu.sync_copy(fc_hbm.at[pl.ds(base, NNZ_PW)], c_v)
        pltpu.sync_copy(fv_hbm.at[pl.ds(base, NNZ_PW)], v_v)

        @pl.loop(0, ZCH, step=16)
        def _(i):
            z_v[pl.ds(i, 16)] = jnp.zeros((16,), jnp.float32)

        @pl.loop(0, SPAN, step=ZCH)
        def _(j):
            pltpu.sync_copy(z_v, s_sh.at[pl.ds(sid * SPAN + j, ZCH)])

        @pl.loop(0, NNZ_PW, step=16)
        def _(i):
            sl = pl.ds(i, 16)
            i_v[sl] = r_v[sl] * D + c_v[sl]

        plsc.subcore_barrier()
        pltpu.sync_copy(v_v, s_sh.at[i_v], add=True)
        plsc.subcore_barrier()
        pltpu.sync_copy(s_sh.at[pl.ds(sid * SPAN, SPAN)],
                        s_hbm.at[pl.ds(cid * (N * D) + sid * SPAN, SPAN)])

    return k(fr, fc, fv)


def _edge_pass(packed, xw):
    @pl.kernel(
        out_type=jax.ShapeDtypeStruct((NC, N, D), jnp.float32),
        mesh=_mesh(),
        scratch_types=(
            [pltpu.VMEM((3, K_E), jnp.int32)] * 3
            + [pltpu.VMEM((K_E, D), jnp.float32)] * 3
            + [pltpu.VMEM((ZR, D), jnp.float32),
               pltpu.VMEM_SHARED((N, D), jnp.float32)]
            + [pltpu.SemaphoreType.DMA] * 6
        ),
    )
    def k(packed_hbm, xw_hbm, o_hbm,
          ib0, ib1, ib2, r0, r1, r2,
          z_v, o_sh, g0, g1, g2, s0, s1, s2):
        cid = lax.axis_index("c")
        sid = lax.axis_index("s")
        wid = cid * NS + sid
        cbase = wid * NCH

        def start(j, ib, r, g):
            pltpu.sync_copy(packed_hbm.at[cbase + j], ib)
            pltpu.async_copy(xw_hbm.at[ib.at[0]], r, g)

        def drain_scatter(ib, r, s):
            pltpu.make_async_copy(r, o_sh.at[ib.at[1]], s).wait()

        def work(ib, r, g, s):
            pltpu.make_async_copy(xw_hbm.at[ib.at[0]], r, g).wait()

            @pl.loop(0, K_E, step=16)
            def _(gg):
                a_vec = lax.bitcast_convert_type(ib[2, pl.ds(gg, 16)],
                                                 jnp.float32)
                for kk in range(16):
                    av = lax.gather(
                        a_vec, jnp.full((16, 1), kk, jnp.int32),
                        lax.GatherDimensionNumbers(
                            offset_dims=(), collapsed_slice_dims=(0,),
                            start_index_map=(0,)),
                        slice_sizes=(1,),
                        mode=lax.GatherScatterMode.PROMISE_IN_BOUNDS)
                    for d in range(8):
                        sl = pl.ds(d * 16, 16)
                        r[gg + kk, sl] = r[gg + kk, sl] * av

            pltpu.async_copy(r, o_sh.at[ib.at[1]], s, add=True)

        start(0, ib0, r0, g0)
        start(1, ib1, r1, g1)
        start(2, ib2, r2, g2)

        @pl.loop(0, ZR)
        def _(i):
            for d in range(8):
                z_v[i, pl.ds(d * 16, 16)] = jnp.zeros((16,), jnp.float32)

        @pl.loop(0, ROWS_PT, step=ZR)
        def _(j):
            pltpu.sync_copy(z_v, o_sh.at[pl.ds(sid * ROWS_PT + j, ZR)])

        @pl.when(sid == 0)
        def _():
            pltpu.sync_copy(z_v.at[pl.ds(0, ROWS_REM)],
                            o_sh.at[pl.ds(NS * ROWS_PT, ROWS_REM)])

        plsc.subcore_barrier()

        @pl.loop(0, NCH, step=3)
        def _(j):
            work(ib0, r0, g0, s0)
            work(ib1, r1, g1, s1)
            drain_scatter(ib0, r0, s0)

            @pl.when(j + 3 < NCH)
            def _():
                start(j + 3, ib0, r0, g0)

            work(ib2, r2, g2, s2)
            drain_scatter(ib1, r1, s1)

            @pl.when(j + 4 < NCH)
            def _():
                start(j + 4, ib1, r1, g1)

            drain_scatter(ib2, r2, s2)

            @pl.when(j + 5 < NCH)
            def _():
                start(j + 5, ib2, r2, g2)

        plsc.subcore_barrier()
        pltpu.sync_copy(o_sh.at[pl.ds(sid * ROWS_PT, ROWS_PT)],
                        o_hbm.at[cid, pl.ds(sid * ROWS_PT, ROWS_PT)])

        @pl.when(sid == 0)
        def _():
            pltpu.sync_copy(o_sh.at[pl.ds(NS * ROWS_PT, ROWS_REM)],
                            o_hbm.at[cid, pl.ds(NS * ROWS_PT, ROWS_REM)])

    return k(packed, xw)


BM = 2000


def _mm(s0, s1, w):
    def body(s0_ref, s1_ref, w_ref, o_ref):
        x = s0_ref[...] + s1_ref[...]
        o_ref[...] = jnp.dot(x, w_ref[...], preferred_element_type=jnp.float32)

    return pl.pallas_call(
        body,
        grid=(N // BM,),
        in_specs=[pl.BlockSpec((BM, D), lambda i: (i, 0)),
                  pl.BlockSpec((BM, D), lambda i: (i, 0)),
                  pl.BlockSpec((D, D), lambda i: (0, 0))],
        out_specs=pl.BlockSpec((BM, D), lambda i: (i, 0)),
        out_shape=jax.ShapeDtypeStruct((N, D), jnp.float32),
    )(s0, s1, w)


def _merge_relu(o0, o1):
    def body(a_ref, b_ref, o_ref):
        o_ref[...] = jnp.maximum(a_ref[...] + b_ref[...], 0.0)

    return pl.pallas_call(
        body,
        grid=(N // BM,),
        in_specs=[pl.BlockSpec((BM, D), lambda i: (i, 0)),
                  pl.BlockSpec((BM, D), lambda i: (i, 0))],
        out_specs=pl.BlockSpec((BM, D), lambda i: (i, 0)),
        out_shape=jax.ShapeDtypeStruct((N, D), jnp.float32),
    )(o0, o1)


def kernel(feat_rows, feat_cols, feat_values, edge_index, adj_values, W):
    pad = NNZ_PAD - NNZ
    fr = jnp.concatenate([feat_rows.astype(jnp.int32),
                          jnp.zeros((pad,), jnp.int32)])
    fc = jnp.concatenate([feat_cols.astype(jnp.int32),
                          jnp.zeros((pad,), jnp.int32)])
    fv = jnp.concatenate([feat_values, jnp.zeros((pad,), jnp.float32)])

    s_parts = _feat_scatter(fr, fc, fv).reshape(NC, N, D)
    xw = _mm(s_parts[0], s_parts[1], W)

    epad = E_PAD - E
    pad_idx = (jnp.arange(epad, dtype=jnp.int32) % N)
    src = jnp.concatenate([edge_index[1].astype(jnp.int32), pad_idx])
    dst = jnp.concatenate([edge_index[0].astype(jnp.int32), pad_idx])
    adj = jnp.concatenate([adj_values, jnp.zeros((epad,), jnp.float32)])
    abits = lax.bitcast_convert_type(adj, jnp.int32)
    packed = (jnp.stack([src, dst, abits])
              .reshape(3, NW * NCH, K_E).transpose(1, 0, 2))

    o_parts = _edge_pass(packed, xw)
    return _merge_relu(o_parts[0], o_parts[1])


# speedup vs baseline: 10.5943x; 1.0841x over previous
import jax
import jax.numpy as jnp
from jax import lax
from jax.experimental import pallas as pl
from jax.experimental.pallas import tpu as pltpu
from jax.experimental.pallas import tpu_sc as plsc

N = 10000
D = 128
NNZ = 100000
E = 320000
NC = 2
NS = 16
NW = NC * NS

NNZ_PW = 3136
NNZ_PAD = NNZ_PW * NW
ZCH = 2000
SPAN = (N * D) // NS
ROWS_PT = 624
ROWS_REM = N - ROWS_PT * NS

K_E = 112
NCH = 90
E_PW = K_E * NCH
E_PAD = E_PW * NW
IB = 3
NBLK = NCH // IB


def _mesh():
    return plsc.VectorSubcoreMesh(core_axis_name="c", subcore_axis_name="s")


def _feat_scatter(fr, fc, fv):
    @pl.kernel(
        out_type=jax.ShapeDtypeStruct((NC * N * D,), jnp.float32),
        mesh=_mesh(),
        scratch_types=[
            pltpu.VMEM((NNZ_PW,), jnp.int32),
            pltpu.VMEM((NNZ_PW,), jnp.int32),
            pltpu.VMEM((NNZ_PW,), jnp.int32),
            pltpu.VMEM((NNZ_PW,), jnp.float32),
            pltpu.VMEM((ZCH,), jnp.float32),
            pltpu.VMEM_SHARED((N * D,), jnp.float32),
        ],
    )
    def k(fr_hbm, fc_hbm, fv_hbm, s_hbm, r_v, c_v, i_v, v_v, z_v, s_sh):
        cid = lax.axis_index("c")
        sid = lax.axis_index("s")
        wid = cid * NS + sid
        base = wid * NNZ_PW
        pltpu.sync_copy(fr_hbm.at[pl.ds(base, NNZ_PW)], r_v)
        pltpu.sync_copy(fc_hbm.at[pl.ds(base, NNZ_PW)], c_v)
        pltpu.sync_copy(fv_hbm.at[pl.ds(base, NNZ_PW)], v_v)

        @pl.loop(0, ZCH, step=16)
        def _(i):
            z_v[pl.ds(i, 16)] = jnp.zeros((16,), jnp.float32)

        @pl.loop(0, SPAN, step=ZCH)
        def _(j):
            pltpu.sync_copy(z_v, s_sh.at[pl.ds(sid * SPAN + j, ZCH)])

        @pl.loop(0, NNZ_PW, step=16)
        def _(i):
            sl = pl.ds(i, 16)
            i_v[sl] = r_v[sl] * D + c_v[sl]

        plsc.subcore_barrier()
        pltpu.sync_copy(v_v, s_sh.at[i_v], add=True)
        plsc.subcore_barrier()
        pltpu.sync_copy(s_sh.at[pl.ds(sid * SPAN, SPAN)],
                        s_hbm.at[pl.ds(cid * (N * D) + sid * SPAN, SPAN)])

    return k(fr, fc, fv)


def _edge_pass(packed, xw):
    @pl.kernel(
        out_type=jax.ShapeDtypeStruct((NC, N, D), jnp.float32),
        mesh=_mesh(),
        scratch_types=(
            [pltpu.VMEM((IB, 3, K_E), jnp.int32)] * 2
            + [pltpu.VMEM((K_E, D), jnp.float32)] * 3
            + [pltpu.VMEM_SHARED((N, D), jnp.float32)]
            + [pltpu.SemaphoreType.DMA] * 7
        ),
    )
    def k(packed_hbm, xw_hbm, o_hbm,
          blkA, blkB, r0, r1, r2,
          o_sh, g0, g1, g2, s0, s1, s2, ibsem):
        cid = lax.axis_index("c")
        sid = lax.axis_index("s")
        wid = cid * NS + sid
        cbase = wid * NCH

        def start(ib, r, g):
            pltpu.async_copy(xw_hbm.at[ib.at[0]], r, g)

        def drain_scatter(ib, r, s):
            pltpu.make_async_copy(r, o_sh.at[ib.at[1]], s).wait()

        def work(ib, r, g, s):
            pltpu.make_async_copy(xw_hbm.at[ib.at[0]], r, g).wait()

            @pl.loop(0, K_E, step=16)
            def _(gg):
                a_vec = lax.bitcast_convert_type(ib[2, pl.ds(gg, 16)],
                                                 jnp.float32)
                for kk in range(16):
                    av = lax.gather(
                        a_vec, jnp.full((16, 1), kk, jnp.int32),
                        lax.GatherDimensionNumbers(
                            offset_dims=(), collapsed_slice_dims=(0,),
                            start_index_map=(0,)),
                        slice_sizes=(1,),
                        mode=lax.GatherScatterMode.PROMISE_IN_BOUNDS)
                    for d in range(8):
                        sl = pl.ds(d * 16, 16)
                        r[gg + kk, sl] = r[gg + kk, sl] * av

            pltpu.async_copy(r, o_sh.at[ib.at[1]], s, add=True)

        @pl.loop(0, K_E)
        def _(i):
            for d in range(8):
                r0[i, pl.ds(d * 16, 16)] = jnp.zeros((16,), jnp.float32)

        for q in range(ROWS_PT // K_E):
            pltpu.sync_copy(r0, o_sh.at[pl.ds(sid * ROWS_PT + q * K_E, K_E)])
        ztail = ROWS_PT % K_E
        pltpu.sync_copy(
            r0.at[pl.ds(0, ztail)],
            o_sh.at[pl.ds(sid * ROWS_PT + ROWS_PT - ztail, ztail)])

        @pl.when(sid == 0)
        def _():
            pltpu.sync_copy(r0.at[pl.ds(0, ROWS_REM)],
                            o_sh.at[pl.ds(NS * ROWS_PT, ROWS_REM)])

        pltpu.sync_copy(packed_hbm.at[pl.ds(cbase, IB)], blkA)
        start(blkA.at[0], r0, g0)
        start(blkA.at[1], r1, g1)
        start(blkA.at[2], r2, g2)

        plsc.subcore_barrier()

        rbuf = ((r0, g0, s0), (r1, g1, s1), (r2, g2, s2))

        @pl.loop(0, NBLK, step=2)
        def _(bb):
            for half in range(2):
                blk, nxt = (blkA, blkB) if half == 0 else (blkB, blkA)
                base = (bb + half) * IB

                @pl.when(base + IB < NCH)
                def _():
                    pltpu.async_copy(
                        packed_hbm.at[pl.ds(cbase + base + IB, IB)],
                        nxt, ibsem)

                for rr in range(IB // 3):
                    p = rr * 3
                    work(blk.at[p + 0], *rbuf[0])
                    work(blk.at[p + 1], *rbuf[1])
                    drain_scatter(blk.at[p + 0], r0, s0)

                    if rr < IB // 3 - 1:
                        start(blk.at[p + 3], r0, g0)
                        work(blk.at[p + 2], *rbuf[2])
                        drain_scatter(blk.at[p + 1], r1, s1)
                        start(blk.at[p + 4], r1, g1)
                        drain_scatter(blk.at[p + 2], r2, s2)
                        start(blk.at[p + 5], r2, g2)
                    else:
                        @pl.when(base + IB < NCH)
                        def _():
                            pltpu.make_async_copy(
                                packed_hbm.at[pl.ds(cbase + base + IB, IB)],
                                nxt, ibsem).wait()
                            start(nxt.at[0], r0, g0)

                        work(blk.at[p + 2], *rbuf[2])
                        drain_scatter(blk.at[p + 1], r1, s1)

                        @pl.when(base + IB < NCH)
                        def _():
                            start(nxt.at[1], r1, g1)

                        drain_scatter(blk.at[p + 2], r2, s2)

                        @pl.when(base + IB < NCH)
                        def _():
                            start(nxt.at[2], r2, g2)

        plsc.subcore_barrier()
        pltpu.sync_copy(o_sh.at[pl.ds(sid * ROWS_PT, ROWS_PT)],
                        o_hbm.at[cid, pl.ds(sid * ROWS_PT, ROWS_PT)])

        @pl.when(sid == 0)
        def _():
            pltpu.sync_copy(o_sh.at[pl.ds(NS * ROWS_PT, ROWS_REM)],
                            o_hbm.at[cid, pl.ds(NS * ROWS_PT, ROWS_REM)])

    return k(packed, xw)


BM = 2000


def _mm(s0, s1, w):
    def body(s0_ref, s1_ref, w_ref, o_ref):
        x = s0_ref[...] + s1_ref[...]
        o_ref[...] = jnp.dot(x, w_ref[...], preferred_element_type=jnp.float32)

    return pl.pallas_call(
        body,
        grid=(N // BM,),
        in_specs=[pl.BlockSpec((BM, D), lambda i: (i, 0)),
                  pl.BlockSpec((BM, D), lambda i: (i, 0)),
                  pl.BlockSpec((D, D), lambda i: (0, 0))],
        out_specs=pl.BlockSpec((BM, D), lambda i: (i, 0)),
        out_shape=jax.ShapeDtypeStruct((N, D), jnp.float32),
    )(s0, s1, w)


def _merge_relu(o0, o1):
    def body(a_ref, b_ref, o_ref):
        o_ref[...] = jnp.maximum(a_ref[...] + b_ref[...], 0.0)

    return pl.pallas_call(
        body,
        grid=(N // BM,),
        in_specs=[pl.BlockSpec((BM, D), lambda i: (i, 0)),
                  pl.BlockSpec((BM, D), lambda i: (i, 0))],
        out_specs=pl.BlockSpec((BM, D), lambda i: (i, 0)),
        out_shape=jax.ShapeDtypeStruct((N, D), jnp.float32),
    )(o0, o1)


def kernel(feat_rows, feat_cols, feat_values, edge_index, adj_values, W):
    pad = NNZ_PAD - NNZ
    fr = jnp.concatenate([feat_rows.astype(jnp.int32),
                          jnp.zeros((pad,), jnp.int32)])
    fc = jnp.concatenate([feat_cols.astype(jnp.int32),
                          jnp.zeros((pad,), jnp.int32)])
    fv = jnp.concatenate([feat_values, jnp.zeros((pad,), jnp.float32)])

    s_parts = _feat_scatter(fr, fc, fv).reshape(NC, N, D)
    xw = _mm(s_parts[0], s_parts[1], W)

    epad = E_PAD - E
    pad_idx = (jnp.arange(epad, dtype=jnp.int32) % N)
    src = jnp.concatenate([edge_index[1].astype(jnp.int32), pad_idx])
    dst = jnp.concatenate([edge_index[0].astype(jnp.int32), pad_idx])
    adj = jnp.concatenate([adj_values, jnp.zeros((epad,), jnp.float32)])
    abits = lax.bitcast_convert_type(adj, jnp.int32)
    packed = (jnp.stack([src, dst, abits])
              .reshape(3, NW * NCH, K_E).transpose(1, 0, 2))

    o_parts = _edge_pass(packed, xw)
    return _merge_relu(o_parts[0], o_parts[1])


# speedup vs baseline: 11.6051x; 1.0954x over previous
import jax
import jax.numpy as jnp
from jax import lax
from jax.experimental import pallas as pl
from jax.experimental.pallas import tpu as pltpu
from jax.experimental.pallas import tpu_sc as plsc

N = 10000
D = 128
NNZ = 100000
E = 320000
NC = 2
NS = 16
NW = NC * NS

NNZ_PW = 6272
NNZ_PAD = NNZ_PW * NS
HALF = (N // 2) * D
TRASH = 2048
SPAN = 39936
SREM = HALF - SPAN * NS
ZSPAN = (HALF + TRASH) // NS
ZCH = 8000
ROWS_PT = 624
ROWS_REM = N - ROWS_PT * NS

K_E = 112
NCH = 90
E_PW = K_E * NCH
E_PAD = E_PW * NW
IB = 3
NBLK = NCH // IB


def _mesh():
    return plsc.VectorSubcoreMesh(core_axis_name="c", subcore_axis_name="s")


def _feat_scatter(fr, fc, fv):
    @pl.kernel(
        out_type=jax.ShapeDtypeStruct((N * D,), jnp.float32),
        mesh=_mesh(),
        scratch_types=[
            pltpu.VMEM((NNZ_PW,), jnp.int32),
            pltpu.VMEM((NNZ_PW,), jnp.int32),
            pltpu.VMEM((NNZ_PW,), jnp.int32),
            pltpu.VMEM((NNZ_PW,), jnp.float32),
            pltpu.VMEM((ZCH,), jnp.float32),
            pltpu.VMEM_SHARED((HALF + TRASH,), jnp.float32),
        ],
    )
    def k(fr_hbm, fc_hbm, fv_hbm, s_hbm, r_v, c_v, i_v, v_v, z_v, s_sh):
        cid = lax.axis_index("c")
        sid = lax.axis_index("s")
        base = sid * NNZ_PW
        pltpu.sync_copy(fr_hbm.at[pl.ds(base, NNZ_PW)], r_v)
        pltpu.sync_copy(fc_hbm.at[pl.ds(base, NNZ_PW)], c_v)
        pltpu.sync_copy(fv_hbm.at[pl.ds(base, NNZ_PW)], v_v)

        @pl.loop(0, ZCH, step=16)
        def _(i):
            z_v[pl.ds(i, 16)] = jnp.zeros((16,), jnp.float32)

        @pl.loop(0, ZSPAN - ZCH + 1, step=ZCH)
        def _(j):
            pltpu.sync_copy(z_v, s_sh.at[pl.ds(sid * ZSPAN + j, ZCH)])

        pltpu.sync_copy(z_v.at[pl.ds(0, ZSPAN % ZCH)],
                        s_sh.at[pl.ds(sid * ZSPAN + ZSPAN - ZSPAN % ZCH,
                                      ZSPAN % ZCH)])

        @pl.loop(0, NNZ_PW, step=16)
        def _(i):
            sl = pl.ds(i, 16)
            flat = r_v[sl] * D + c_v[sl]
            loc = flat - cid * HALF
            inb = (loc >= 0) & (loc < HALF)
            i_v[sl] = jnp.where(inb, loc,
                                HALF + (flat & (TRASH - 1)))

        plsc.subcore_barrier()
        pltpu.sync_copy(v_v, s_sh.at[i_v], add=True)
        plsc.subcore_barrier()
        pltpu.sync_copy(s_sh.at[pl.ds(sid * SPAN, SPAN)],
                        s_hbm.at[pl.ds(cid * HALF + sid * SPAN, SPAN)])

        @pl.when(sid == 0)
        def _():
            pltpu.sync_copy(
                s_sh.at[pl.ds(NS * SPAN, SREM)],
                s_hbm.at[pl.ds(cid * HALF + NS * SPAN, SREM)])

    return k(fr, fc, fv)


def _edge_pass(packed, xw):
    @pl.kernel(
        out_type=jax.ShapeDtypeStruct((NC, N, D), jnp.float32),
        mesh=_mesh(),
        scratch_types=(
            [pltpu.VMEM((IB, 3, K_E), jnp.int32)] * 2
            + [pltpu.VMEM((K_E, D), jnp.float32)] * 3
            + [pltpu.VMEM_SHARED((N, D), jnp.float32)]
            + [pltpu.SemaphoreType.DMA] * 7
        ),
    )
    def k(packed_hbm, xw_hbm, o_hbm,
          blkA, blkB, r0, r1, r2,
          o_sh, g0, g1, g2, s0, s1, s2, ibsem):
        cid = lax.axis_index("c")
        sid = lax.axis_index("s")
        wid = cid * NS + sid
        cbase = wid * NCH

        def start(ib, r, g):
            pltpu.async_copy(xw_hbm.at[ib.at[0]], r, g)

        def drain_scatter(ib, r, s):
            pltpu.make_async_copy(r, o_sh.at[ib.at[1]], s).wait()

        def work(ib, r, g, s):
            pltpu.make_async_copy(xw_hbm.at[ib.at[0]], r, g).wait()

            @pl.loop(0, K_E, step=16)
            def _(gg):
                a_vec = lax.bitcast_convert_type(ib[2, pl.ds(gg, 16)],
                                                 jnp.float32)
                for kk in range(16):
                    av = lax.gather(
                        a_vec, jnp.full((16, 1), kk, jnp.int32),
                        lax.GatherDimensionNumbers(
                            offset_dims=(), collapsed_slice_dims=(0,),
                            start_index_map=(0,)),
                        slice_sizes=(1,),
                        mode=lax.GatherScatterMode.PROMISE_IN_BOUNDS)
                    for d in range(8):
                        sl = pl.ds(d * 16, 16)
                        r[gg + kk, sl] = r[gg + kk, sl] * av

            pltpu.async_copy(r, o_sh.at[ib.at[1]], s, add=True)

        @pl.loop(0, K_E)
        def _(i):
            for d in range(8):
                r0[i, pl.ds(d * 16, 16)] = jnp.zeros((16,), jnp.float32)

        for q in range(ROWS_PT // K_E):
            pltpu.sync_copy(r0, o_sh.at[pl.ds(sid * ROWS_PT + q * K_E, K_E)])
        ztail = ROWS_PT % K_E
        pltpu.sync_copy(
            r0.at[pl.ds(0, ztail)],
            o_sh.at[pl.ds(sid * ROWS_PT + ROWS_PT - ztail, ztail)])

        @pl.when(sid == 0)
        def _():
            pltpu.sync_copy(r0.at[pl.ds(0, ROWS_REM)],
                            o_sh.at[pl.ds(NS * ROWS_PT, ROWS_REM)])

        pltpu.sync_copy(packed_hbm.at[pl.ds(cbase, IB)], blkA)
        start(blkA.at[0], r0, g0)
        start(blkA.at[1], r1, g1)
        start(blkA.at[2], r2, g2)

        plsc.subcore_barrier()

        rbuf = ((r0, g0, s0), (r1, g1, s1), (r2, g2, s2))

        @pl.loop(0, NBLK, step=2)
        def _(bb):
            for half in range(2):
                blk, nxt = (blkA, blkB) if half == 0 else (blkB, blkA)
                base = (bb + half) * IB

                @pl.when(base + IB < NCH)
                def _():
                    pltpu.async_copy(
                        packed_hbm.at[pl.ds(cbase + base + IB, IB)],
                        nxt, ibsem)

                for rr in range(IB // 3):
                    p = rr * 3
                    work(blk.at[p + 0], *rbuf[0])
                    work(blk.at[p + 1], *rbuf[1])
                    drain_scatter(blk.at[p + 0], r0, s0)

                    if rr < IB // 3 - 1:
                        start(blk.at[p + 3], r0, g0)
                        work(blk.at[p + 2], *rbuf[2])
                        drain_scatter(blk.at[p + 1], r1, s1)
                        start(blk.at[p + 4], r1, g1)
                        drain_scatter(blk.at[p + 2], r2, s2)
                        start(blk.at[p + 5], r2, g2)
                    else:
                        @pl.when(base + IB < NCH)
                        def _():
                            pltpu.make_async_copy(
                                packed_hbm.at[pl.ds(cbase + base + IB, IB)],
                                nxt, ibsem).wait()
                            start(nxt.at[0], r0, g0)

                        work(blk.at[p + 2], *rbuf[2])
                        drain_scatter(blk.at[p + 1], r1, s1)

                        @pl.when(base + IB < NCH)
                        def _():
                            start(nxt.at[1], r1, g1)

                        drain_scatter(blk.at[p + 2], r2, s2)

                        @pl.when(base + IB < NCH)
                        def _():
                            start(nxt.at[2], r2, g2)

        plsc.subcore_barrier()
        pltpu.sync_copy(o_sh.at[pl.ds(sid * ROWS_PT, ROWS_PT)],
                        o_hbm.at[cid, pl.ds(sid * ROWS_PT, ROWS_PT)])

        @pl.when(sid == 0)
        def _():
            pltpu.sync_copy(o_sh.at[pl.ds(NS * ROWS_PT, ROWS_REM)],
                            o_hbm.at[cid, pl.ds(NS * ROWS_PT, ROWS_REM)])

    return k(packed, xw)


BM = 2000


def _mm_relu(p0, p1, w):
    def body(p0_ref, p1_ref, w_ref, o_ref):
        x = p0_ref[...] + p1_ref[...]
        o_ref[...] = jnp.maximum(
            jnp.dot(x, w_ref[...], preferred_element_type=jnp.float32), 0.0)

    return pl.pallas_call(
        body,
        grid=(N // BM,),
        in_specs=[pl.BlockSpec((BM, D), lambda i: (i, 0)),
                  pl.BlockSpec((BM, D), lambda i: (i, 0)),
                  pl.BlockSpec((D, D), lambda i: (0, 0))],
        out_specs=pl.BlockSpec((BM, D), lambda i: (i, 0)),
        out_shape=jax.ShapeDtypeStruct((N, D), jnp.float32),
    )(p0, p1, w)


def kernel(feat_rows, feat_cols, feat_values, edge_index, adj_values, W):
    pad = NNZ_PAD - NNZ
    fr = jnp.concatenate([feat_rows.astype(jnp.int32),
                          jnp.zeros((pad,), jnp.int32)])
    fc = jnp.concatenate([feat_cols.astype(jnp.int32),
                          jnp.zeros((pad,), jnp.int32)])
    fv = jnp.concatenate([feat_values, jnp.zeros((pad,), jnp.float32)])

    s = _feat_scatter(fr, fc, fv).reshape(N, D)

    epad = E_PAD - E
    pad_idx = (jnp.arange(epad, dtype=jnp.int32) % N)
    src = jnp.concatenate([edge_index[1].astype(jnp.int32), pad_idx])
    dst = jnp.concatenate([edge_index[0].astype(jnp.int32), pad_idx])
    adj = jnp.concatenate([adj_values, jnp.zeros((epad,), jnp.float32)])
    abits = lax.bitcast_convert_type(adj, jnp.int32)
    packed = (jnp.stack([src, dst, abits])
              .reshape(3, NW * NCH, K_E).transpose(1, 0, 2))

    p_parts = _edge_pass(packed, s)
    return _mm_relu(p_parts[0], p_parts[1], W)
